# baseline (speedup 1.0000x reference)
import functools

import jax
import jax.numpy as jnp
from jax import lax
from jax.experimental import pallas as pl
from jax.experimental.pallas import tpu as pltpu
from jax.experimental.pallas import tpu_sc as plsc

_NMS_THR = 0.3
_MAX_OUT = 320


def _nms_body(scores_ref, x1_ref, y1_ref, x2_ref, y2_ref, keep_ref, s_ref, a_ref):
    x1 = x1_ref[...]
    y1 = y1_ref[...]
    x2 = x2_ref[...]
    y2 = y2_ref[...]
    areas = jnp.maximum(x2 - x1, 0.0) * jnp.maximum(y2 - y1, 0.0)
    a_ref[...] = areas
    s_ref[...] = scores_ref[...]
    rows, m_cols = s_ref.shape
    col = lax.broadcasted_iota(jnp.int32, (rows, m_cols), 1)
    row = lax.broadcasted_iota(jnp.int32, (rows, m_cols), 0)
    gidx = (col + jnp.where(row >= 4, m_cols, 0)).astype(jnp.float32)
    nbig = jnp.float32(2 * m_cols)

    def both(h):
        return jnp.concatenate([h, h], axis=0)

    def body(_):
        s = s_ref[...]
        a = a_ref[...]
        m = jnp.max(s, axis=1, keepdims=True)
        mh = jnp.maximum(m[:4], m[4:])
        m8 = both(mh)
        active = m8 > -0.5
        im = jnp.min(jnp.where(s == m8, gidx, nbig), axis=1, keepdims=True)
        idx8 = both(jnp.minimum(im[:4], im[4:]))
        cand = (gidx == idx8) & active
        cf = cand.astype(jnp.float32)

        def pick(arr):
            t = jnp.sum(arr * cf, axis=1, keepdims=True)
            return both(t[:4] + t[4:])

        cx1 = pick(x1)
        cy1 = pick(y1)
        cx2 = pick(x2)
        cy2 = pick(y2)
        ca = pick(a)
        xx1 = jnp.maximum(x1, cx1)
        yy1 = jnp.maximum(y1, cy1)
        xx2 = jnp.minimum(x2, cx2)
        yy2 = jnp.minimum(y2, cy2)
        inter = jnp.maximum(xx2 - xx1, 0.0) * jnp.maximum(yy2 - yy1, 0.0)
        union = ca + a - inter
        iou = inter / jnp.maximum(union, 1e-9)
        supp = (iou > _NMS_THR) & active
        s_new = jnp.where(cand, -2.0, jnp.where(supp, -1.0, s))
        s_ref[...] = s_new
        return jnp.any(s_new > -0.5)

    lax.while_loop(lambda c: c, body, jnp.any(s_ref[...] > -0.5))
    keep_ref[...] = (s_ref[...] == -2.0).astype(jnp.float32)


def _compact_body(keep_hbm, cls_hbm, x1_hbm, y1_hbm, x2_hbm, y2_hbm, sc_hbm,
                  oc_hbm, o1_hbm, o2_hbm, o3_hbm, o4_hbm, os_hbm,
                  k_s, c_s, x1_s, y1_s, x2_s, y2_s, s_s,
                  oc_s, o1_s, o2_s, o3_s, o4_s, os_s):
    nb = keep_hbm.shape[0]
    npad = keep_hbm.shape[1]
    wid = lax.axis_index("s") * 2 + lax.axis_index("c")

    @pl.when(wid < nb)
    def _():
        pltpu.sync_copy(keep_hbm.at[wid], k_s)
        pltpu.sync_copy(cls_hbm.at[wid], c_s)
        pltpu.sync_copy(x1_hbm.at[wid], x1_s)
        pltpu.sync_copy(y1_hbm.at[wid], y1_s)
        pltpu.sync_copy(x2_hbm.at[wid], x2_s)
        pltpu.sync_copy(y2_hbm.at[wid], y2_s)
        pltpu.sync_copy(sc_hbm.at[wid], s_s)

        outs = (oc_s, o1_s, o2_s, o3_s, o4_s, os_s)
        srcs = (c_s, x1_s, y1_s, x2_s, y2_s, s_s)

        def zero(i, _):
            z = jnp.zeros((16,), jnp.float32)
            for oref in outs:
                oref[pl.ds(i * 16, 16)] = z
            return 0

        lax.fori_loop(0, _MAX_OUT // 16, zero, 0)

        def step(i, base):
            kv = k_s[pl.ds(i * 16, 16)]
            ci = plsc.cumsum(kv)
            pos = base + ci.astype(jnp.int32) - 1
            msk = (kv > 0.5) & (pos < _MAX_OUT)
            for src, dst in zip(srcs, outs):
                plsc.store_scatter(dst, [pos], src[pl.ds(i * 16, 16)], mask=msk)
            return base + jnp.sum(kv).astype(jnp.int32)

        lax.fori_loop(0, npad // 16, step, jnp.int32(0))

        pltpu.sync_copy(oc_s, oc_hbm.at[wid])
        pltpu.sync_copy(o1_s, o1_hbm.at[wid])
        pltpu.sync_copy(o2_s, o2_hbm.at[wid])
        pltpu.sync_copy(o3_s, o3_hbm.at[wid])
        pltpu.sync_copy(o4_s, o4_hbm.at[wid])
        pltpu.sync_copy(os_s, os_hbm.at[wid])


@functools.partial(jax.jit, static_argnums=(5,))
def _run_nms(scores8, x18, y18, x28, y28, m_cols):
    return pl.pallas_call(
        _nms_body,
        out_shape=jax.ShapeDtypeStruct((8, m_cols), jnp.float32),
        scratch_shapes=[
            pltpu.VMEM((8, m_cols), jnp.float32),
            pltpu.VMEM((8, m_cols), jnp.float32),
        ],
    )(scores8, x18, y18, x28, y28)


@functools.partial(jax.jit, static_argnums=(7, 8))
def _run_compact(keep, cls_a, x1, y1, x2, y2, sc_a, nb, npad):
    mesh = plsc.VectorSubcoreMesh(core_axis_name="c", subcore_axis_name="s")
    out_type = [jax.ShapeDtypeStruct((nb, _MAX_OUT), jnp.float32)] * 6
    scratch = [pltpu.VMEM((npad,), jnp.float32)] * 7 + \
              [pltpu.VMEM((_MAX_OUT,), jnp.float32)] * 6
    return pl.kernel(
        _compact_body,
        out_type=out_type,
        mesh=mesh,
        scratch_types=scratch,
        compiler_params=pltpu.CompilerParams(needs_layout_passes=False),
    )(keep, cls_a, x1, y1, x2, y2, sc_a)


def kernel(cls_proposals, gt_classes, box_proposals, gt_boxes, proposal_scores):
    nb = gt_boxes.shape[0]
    cls_all = jnp.concatenate([gt_classes, cls_proposals], axis=1)
    box_all = jnp.concatenate([gt_boxes, box_proposals], axis=1)
    sc_all = jnp.concatenate([gt_classes, proposal_scores], axis=1)
    n = box_all.shape[1]
    npad = ((n + 511) // 512) * 512

    x1 = box_all[:, :, 0]
    y1 = box_all[:, :, 1]
    x2 = box_all[:, :, 2]
    y2 = box_all[:, :, 3]

    m_cols = npad // 2

    def fold(arr, value):
        out = jnp.full((nb, npad), value, jnp.float32)
        out = out.at[:, :n].set(arr)
        return jnp.concatenate([out[:, :m_cols], out[:, m_cols:]], axis=0)

    scores8 = fold(sc_all, -1.0)
    x18 = fold(x1, 0.0)
    y18 = fold(y1, 0.0)
    x28 = fold(x2, 0.0)
    y28 = fold(y2, 0.0)

    keep8 = _run_nms(scores8, x18, y18, x28, y28, m_cols)
    keep = jnp.concatenate([keep8[:nb], keep8[nb:]], axis=1)

    def pad_cols(arr):
        return jnp.pad(arr, ((0, 0), (0, npad - n)))

    oc, o1, o2, o3, o4, osc = _run_compact(
        keep, pad_cols(cls_all), pad_cols(x1), pad_cols(y1), pad_cols(x2),
        pad_cols(y2), pad_cols(sc_all), nb, npad)

    outb = jnp.stack([o1, o2, o3, o4], axis=-1)
    return oc, outb, osc


# speedup vs baseline: 2.2157x; 2.2157x over previous
import functools

import jax
import jax.numpy as jnp
from jax import lax
from jax.experimental import pallas as pl
from jax.experimental.pallas import tpu as pltpu
from jax.experimental.pallas import tpu_sc as plsc

_NMS_THR = 0.3
_MAX_OUT = 320


def _nms_body(scores_ref, x1_ref, y1_ref, x2_ref, y2_ref, keep_ref, s_ref, a_ref):
    n_b = scores_ref.shape[0]
    rows, fcols = scores_ref.shape[1], scores_ref.shape[2]
    for b in range(n_b):
        x1 = x1_ref[b]
        y1 = y1_ref[b]
        x2 = x2_ref[b]
        y2 = y2_ref[b]
        a_ref[b] = jnp.maximum(x2 - x1, 0.0) * jnp.maximum(y2 - y1, 0.0)
        s_ref[b] = scores_ref[b]
    col = lax.broadcasted_iota(jnp.int32, (rows, fcols), 1)
    row = lax.broadcasted_iota(jnp.int32, (rows, fcols), 0)
    gidx = (col + fcols * row).astype(jnp.float32)
    nbig = jnp.float32(rows * fcols)

    def red(op, arr):
        return op(op(arr, axis=1, keepdims=True), axis=0, keepdims=True)

    def chain(b, m1):
        s = s_ref[b]
        a = a_ref[b]
        x1 = x1_ref[b]
        y1 = y1_ref[b]
        x2 = x2_ref[b]
        y2 = y2_ref[b]
        act1 = m1 > -0.5
        idx1 = red(jnp.min, jnp.where(s == m1, gidx, nbig))
        cand1 = (gidx == idx1) & act1

        s_excl = jnp.where(cand1, -3.0, s)
        m2 = red(jnp.max, s_excl)
        act2 = m2 > -0.5
        idx2 = red(jnp.min, jnp.where(s_excl == m2, gidx, nbig))
        cand2 = (gidx == idx2) & act2

        c1 = cand1.astype(jnp.float32)
        c2 = cand2.astype(jnp.float32)
        ax1 = red(jnp.sum, x1 * c1)
        ay1 = red(jnp.sum, y1 * c1)
        ax2 = red(jnp.sum, x2 * c1)
        ay2 = red(jnp.sum, y2 * c1)
        aa = red(jnp.sum, a * c1)
        bx1 = red(jnp.sum, x1 * c2)
        by1 = red(jnp.sum, y1 * c2)
        bx2 = red(jnp.sum, x2 * c2)
        by2 = red(jnp.sum, y2 * c2)
        ba = red(jnp.sum, a * c2)

        pxx1 = jnp.maximum(ax1, bx1)
        pyy1 = jnp.maximum(ay1, by1)
        pxx2 = jnp.minimum(ax2, bx2)
        pyy2 = jnp.minimum(ay2, by2)
        pinter = jnp.maximum(pxx2 - pxx1, 0.0) * jnp.maximum(pyy2 - pyy1, 0.0)
        piou = pinter / jnp.maximum(aa + ba - pinter, 1e-9)
        okb = act2 & jnp.logical_not(piou > _NMS_THR)

        xx1a = jnp.maximum(x1, ax1)
        yy1a = jnp.maximum(y1, ay1)
        xx2a = jnp.minimum(x2, ax2)
        yy2a = jnp.minimum(y2, ay2)
        intera = jnp.maximum(xx2a - xx1a, 0.0) * jnp.maximum(yy2a - yy1a, 0.0)
        ioua = intera / jnp.maximum(aa + a - intera, 1e-9)
        xx1b = jnp.maximum(x1, bx1)
        yy1b = jnp.maximum(y1, by1)
        xx2b = jnp.minimum(x2, bx2)
        yy2b = jnp.minimum(y2, by2)
        interb = jnp.maximum(xx2b - xx1b, 0.0) * jnp.maximum(yy2b - yy1b, 0.0)
        ioub = interb / jnp.maximum(ba + a - interb, 1e-9)

        supp = ((ioua > _NMS_THR) & act1) | ((ioub > _NMS_THR) & okb)
        commit = cand1 | (cand2 & okb)
        s_new = jnp.where(commit, -2.0, jnp.where(supp, -1.0, s))
        s_ref[b] = s_new
        return red(jnp.max, s_new)

    def body(carry):
        return tuple(chain(b, carry[b]) for b in range(n_b))

    def cond(carry):
        m = carry[0]
        for b in range(1, n_b):
            m = jnp.maximum(m, carry[b])
        return m[0, 0] > -0.5

    init = tuple(red(jnp.max, s_ref[b]) for b in range(n_b))
    lax.while_loop(cond, body, init)
    for b in range(n_b):
        keep_ref[b] = (s_ref[b] == -2.0).astype(jnp.float32)


def _compact_body(keep_hbm, cls_hbm, x1_hbm, y1_hbm, x2_hbm, y2_hbm, sc_hbm,
                  oc_hbm, o1_hbm, o2_hbm, o3_hbm, o4_hbm, os_hbm,
                  k_s, c_s, x1_s, y1_s, x2_s, y2_s, s_s,
                  oc_s, o1_s, o2_s, o3_s, o4_s, os_s):
    nb = keep_hbm.shape[0]
    npad = keep_hbm.shape[1]
    wid = lax.axis_index("s") * 2 + lax.axis_index("c")

    @pl.when(wid < nb)
    def _():
        pltpu.sync_copy(keep_hbm.at[wid], k_s)
        pltpu.sync_copy(cls_hbm.at[wid], c_s)
        pltpu.sync_copy(x1_hbm.at[wid], x1_s)
        pltpu.sync_copy(y1_hbm.at[wid], y1_s)
        pltpu.sync_copy(x2_hbm.at[wid], x2_s)
        pltpu.sync_copy(y2_hbm.at[wid], y2_s)
        pltpu.sync_copy(sc_hbm.at[wid], s_s)

        outs = (oc_s, o1_s, o2_s, o3_s, o4_s, os_s)
        srcs = (c_s, x1_s, y1_s, x2_s, y2_s, s_s)

        def zero(i, _):
            z = jnp.zeros((16,), jnp.float32)
            for oref in outs:
                oref[pl.ds(i * 16, 16)] = z
            return 0

        lax.fori_loop(0, _MAX_OUT // 16, zero, 0)

        def step(i, base):
            kv = k_s[pl.ds(i * 16, 16)]
            ci = plsc.cumsum(kv)
            pos = base + ci.astype(jnp.int32) - 1
            msk = (kv > 0.5) & (pos < _MAX_OUT)
            for src, dst in zip(srcs, outs):
                plsc.store_scatter(dst, [pos], src[pl.ds(i * 16, 16)], mask=msk)
            return base + jnp.sum(kv).astype(jnp.int32)

        lax.fori_loop(0, npad // 16, step, jnp.int32(0))

        pltpu.sync_copy(oc_s, oc_hbm.at[wid])
        pltpu.sync_copy(o1_s, o1_hbm.at[wid])
        pltpu.sync_copy(o2_s, o2_hbm.at[wid])
        pltpu.sync_copy(o3_s, o3_hbm.at[wid])
        pltpu.sync_copy(o4_s, o4_hbm.at[wid])
        pltpu.sync_copy(os_s, os_hbm.at[wid])


@functools.partial(jax.jit, static_argnums=(5,))
def _run_nms(scores8, x18, y18, x28, y28, shape3):
    return pl.pallas_call(
        _nms_body,
        out_shape=jax.ShapeDtypeStruct(shape3, jnp.float32),
        scratch_shapes=[
            pltpu.VMEM(shape3, jnp.float32),
            pltpu.VMEM(shape3, jnp.float32),
        ],
    )(scores8, x18, y18, x28, y28)


@functools.partial(jax.jit, static_argnums=(7, 8))
def _run_compact(keep, cls_a, x1, y1, x2, y2, sc_a, nb, npad):
    mesh = plsc.VectorSubcoreMesh(core_axis_name="c", subcore_axis_name="s")
    out_type = [jax.ShapeDtypeStruct((nb, _MAX_OUT), jnp.float32)] * 6
    scratch = [pltpu.VMEM((npad,), jnp.float32)] * 7 + \
              [pltpu.VMEM((_MAX_OUT,), jnp.float32)] * 6
    return pl.kernel(
        _compact_body,
        out_type=out_type,
        mesh=mesh,
        scratch_types=scratch,
        compiler_params=pltpu.CompilerParams(needs_layout_passes=False),
    )(keep, cls_a, x1, y1, x2, y2, sc_a)


def kernel(cls_proposals, gt_classes, box_proposals, gt_boxes, proposal_scores):
    nb = gt_boxes.shape[0]
    cls_all = jnp.concatenate([gt_classes, cls_proposals], axis=1)
    box_all = jnp.concatenate([gt_boxes, box_proposals], axis=1)
    sc_all = jnp.concatenate([gt_classes, proposal_scores], axis=1)
    n = box_all.shape[1]
    npad = ((n + 511) // 512) * 512

    x1 = box_all[:, :, 0]
    y1 = box_all[:, :, 1]
    x2 = box_all[:, :, 2]
    y2 = box_all[:, :, 3]

    fcols = npad // 8
    shape3 = (nb, 8, fcols)

    def fold(arr, value):
        out = jnp.full((nb, npad), value, jnp.float32)
        out = out.at[:, :n].set(arr)
        return out.reshape(shape3)

    scores8 = fold(sc_all, -1.0)
    x18 = fold(x1, 0.0)
    y18 = fold(y1, 0.0)
    x28 = fold(x2, 0.0)
    y28 = fold(y2, 0.0)

    keep = _run_nms(scores8, x18, y18, x28, y28, shape3).reshape(nb, npad)

    def pad_cols(arr):
        return jnp.pad(arr, ((0, 0), (0, npad - n)))

    oc, o1, o2, o3, o4, osc = _run_compact(
        keep, pad_cols(cls_all), pad_cols(x1), pad_cols(y1), pad_cols(x2),
        pad_cols(y2), pad_cols(sc_all), nb, npad)

    outb = jnp.stack([o1, o2, o3, o4], axis=-1)
    return oc, outb, osc


# speedup vs baseline: 2.3740x; 1.0714x over previous
import functools

import jax
import jax.numpy as jnp
from jax import lax
from jax.experimental import pallas as pl
from jax.experimental.pallas import tpu as pltpu
from jax.experimental.pallas import tpu_sc as plsc

_NMS_THR = 0.3
_MAX_OUT = 320
_SPEC_K = 8


def _nms_body(scores_ref, x1_ref, y1_ref, x2_ref, y2_ref, keep_ref, s_ref, a_ref):
    n_b = scores_ref.shape[0]
    rows, fcols = scores_ref.shape[1], scores_ref.shape[2]
    for b in range(n_b):
        x1 = x1_ref[b]
        y1 = y1_ref[b]
        x2 = x2_ref[b]
        y2 = y2_ref[b]
        a_ref[b] = jnp.maximum(x2 - x1, 0.0) * jnp.maximum(y2 - y1, 0.0)
        s_ref[b] = scores_ref[b]
    col = lax.broadcasted_iota(jnp.int32, (rows, fcols), 1)
    row = lax.broadcasted_iota(jnp.int32, (rows, fcols), 0)
    gidx = (col + fcols * row).astype(jnp.float32)
    nbig = jnp.float32(rows * fcols)

    def red(op, arr):
        return op(op(arr, axis=1, keepdims=True), axis=0, keepdims=True)

    def chain(b, m1):
        s = s_ref[b]
        a = a_ref[b]
        x1 = x1_ref[b]
        y1 = y1_ref[b]
        x2 = x2_ref[b]
        y2 = y2_ref[b]

        cands = []
        s_cur = s
        m = m1
        for g in range(_SPEC_K):
            act = m > -0.5
            idx = red(jnp.min, jnp.where(s_cur == m, gidx, nbig))
            cand = (gidx == idx) & act
            cands.append((cand, act))
            s_cur = jnp.where(cand, -3.0, s_cur)
            if g + 1 < _SPEC_K:
                m = red(jnp.max, s_cur)

        coords = []
        for cand, act in cands:
            cf = cand.astype(jnp.float32)
            coords.append((red(jnp.sum, x1 * cf), red(jnp.sum, y1 * cf),
                           red(jnp.sum, x2 * cf), red(jnp.sum, y2 * cf),
                           red(jnp.sum, a * cf)))

        def pair_iou(i, j):
            ix1, iy1, ix2, iy2, ia = coords[i]
            jx1, jy1, jx2, jy2, ja = coords[j]
            w = jnp.maximum(jnp.minimum(ix2, jx2) - jnp.maximum(ix1, jx1), 0.0)
            h = jnp.maximum(jnp.minimum(iy2, jy2) - jnp.maximum(iy1, jy1), 0.0)
            inter = w * h
            return inter / jnp.maximum(ia + ja - inter, 1e-9)

        commit = [cands[0][1]]
        for g in range(1, _SPEC_K):
            sup = commit[0] & (pair_iou(0, g) > _NMS_THR)
            for h in range(1, g):
                sup = sup | (commit[h] & (pair_iou(h, g) > _NMS_THR))
            commit.append(cands[g][1] & jnp.logical_not(sup))

        supp = None
        commit_mask = None
        for g in range(_SPEC_K):
            gx1, gy1, gx2, gy2, ga = coords[g]
            w = jnp.maximum(jnp.minimum(x2, gx2) - jnp.maximum(x1, gx1), 0.0)
            h = jnp.maximum(jnp.minimum(y2, gy2) - jnp.maximum(y1, gy1), 0.0)
            inter = w * h
            iou = inter / jnp.maximum(ga + a - inter, 1e-9)
            sg = (iou > _NMS_THR) & commit[g]
            cg = cands[g][0] & commit[g]
            supp = sg if supp is None else (supp | sg)
            commit_mask = cg if commit_mask is None else (commit_mask | cg)

        s_new = jnp.where(commit_mask, -2.0, jnp.where(supp, -1.0, s))
        s_ref[b] = s_new
        return red(jnp.max, s_new)

    def body(carry):
        return tuple(chain(b, carry[b]) for b in range(n_b))

    def cond(carry):
        m = carry[0]
        for b in range(1, n_b):
            m = jnp.maximum(m, carry[b])
        return m[0, 0] > -0.5

    init = tuple(red(jnp.max, s_ref[b]) for b in range(n_b))
    lax.while_loop(cond, body, init)
    for b in range(n_b):
        keep_ref[b] = (s_ref[b] == -2.0).astype(jnp.float32)


def _compact_body(keep_hbm, cls_hbm, x1_hbm, y1_hbm, x2_hbm, y2_hbm, sc_hbm,
                  oc_hbm, o1_hbm, o2_hbm, o3_hbm, o4_hbm, os_hbm,
                  k_s, c_s, x1_s, y1_s, x2_s, y2_s, s_s,
                  oc_s, o1_s, o2_s, o3_s, o4_s, os_s):
    nb = keep_hbm.shape[0]
    npad = keep_hbm.shape[1]
    wid = lax.axis_index("s") * 2 + lax.axis_index("c")

    @pl.when(wid < nb)
    def _():
        pltpu.sync_copy(keep_hbm.at[wid], k_s)
        pltpu.sync_copy(cls_hbm.at[wid], c_s)
        pltpu.sync_copy(x1_hbm.at[wid], x1_s)
        pltpu.sync_copy(y1_hbm.at[wid], y1_s)
        pltpu.sync_copy(x2_hbm.at[wid], x2_s)
        pltpu.sync_copy(y2_hbm.at[wid], y2_s)
        pltpu.sync_copy(sc_hbm.at[wid], s_s)

        outs = (oc_s, o1_s, o2_s, o3_s, o4_s, os_s)
        srcs = (c_s, x1_s, y1_s, x2_s, y2_s, s_s)

        def zero(i, _):
            z = jnp.zeros((16,), jnp.float32)
            for oref in outs:
                oref[pl.ds(i * 16, 16)] = z
            return 0

        lax.fori_loop(0, _MAX_OUT // 16, zero, 0)

        def step(i, base):
            kv = k_s[pl.ds(i * 16, 16)]
            ci = plsc.cumsum(kv)
            pos = base + ci.astype(jnp.int32) - 1
            msk = (kv > 0.5) & (pos < _MAX_OUT)
            for src, dst in zip(srcs, outs):
                plsc.store_scatter(dst, [pos], src[pl.ds(i * 16, 16)], mask=msk)
            return base + jnp.sum(kv).astype(jnp.int32)

        lax.fori_loop(0, npad // 16, step, jnp.int32(0))

        pltpu.sync_copy(oc_s, oc_hbm.at[wid])
        pltpu.sync_copy(o1_s, o1_hbm.at[wid])
        pltpu.sync_copy(o2_s, o2_hbm.at[wid])
        pltpu.sync_copy(o3_s, o3_hbm.at[wid])
        pltpu.sync_copy(o4_s, o4_hbm.at[wid])
        pltpu.sync_copy(os_s, os_hbm.at[wid])


@functools.partial(jax.jit, static_argnums=(5,))
def _run_nms(scores8, x18, y18, x28, y28, shape3):
    return pl.pallas_call(
        _nms_body,
        out_shape=jax.ShapeDtypeStruct(shape3, jnp.float32),
        scratch_shapes=[
            pltpu.VMEM(shape3, jnp.float32),
            pltpu.VMEM(shape3, jnp.float32),
        ],
    )(scores8, x18, y18, x28, y28)


@functools.partial(jax.jit, static_argnums=(7, 8))
def _run_compact(keep, cls_a, x1, y1, x2, y2, sc_a, nb, npad):
    mesh = plsc.VectorSubcoreMesh(core_axis_name="c", subcore_axis_name="s")
    out_type = [jax.ShapeDtypeStruct((nb, _MAX_OUT), jnp.float32)] * 6
    scratch = [pltpu.VMEM((npad,), jnp.float32)] * 7 + \
              [pltpu.VMEM((_MAX_OUT,), jnp.float32)] * 6
    return pl.kernel(
        _compact_body,
        out_type=out_type,
        mesh=mesh,
        scratch_types=scratch,
        compiler_params=pltpu.CompilerParams(needs_layout_passes=False),
    )(keep, cls_a, x1, y1, x2, y2, sc_a)


def kernel(cls_proposals, gt_classes, box_proposals, gt_boxes, proposal_scores):
    nb = gt_boxes.shape[0]
    cls_all = jnp.concatenate([gt_classes, cls_proposals], axis=1)
    box_all = jnp.concatenate([gt_boxes, box_proposals], axis=1)
    sc_all = jnp.concatenate([gt_classes, proposal_scores], axis=1)
    n = box_all.shape[1]
    npad = ((n + 511) // 512) * 512

    x1 = box_all[:, :, 0]
    y1 = box_all[:, :, 1]
    x2 = box_all[:, :, 2]
    y2 = box_all[:, :, 3]

    fcols = npad // 8
    shape3 = (nb, 8, fcols)

    def fold(arr, value):
        out = jnp.full((nb, npad), value, jnp.float32)
        out = out.at[:, :n].set(arr)
        return out.reshape(shape3)

    scores8 = fold(sc_all, -1.0)
    x18 = fold(x1, 0.0)
    y18 = fold(y1, 0.0)
    x28 = fold(x2, 0.0)
    y28 = fold(y2, 0.0)

    keep = _run_nms(scores8, x18, y18, x28, y28, shape3).reshape(nb, npad)

    def pad_cols(arr):
        return jnp.pad(arr, ((0, 0), (0, npad - n)))

    oc, o1, o2, o3, o4, osc = _run_compact(
        keep, pad_cols(cls_all), pad_cols(x1), pad_cols(y1), pad_cols(x2),
        pad_cols(y2), pad_cols(sc_all), nb, npad)

    outb = jnp.stack([o1, o2, o3, o4], axis=-1)
    return oc, outb, osc


# speedup vs baseline: 2.3908x; 1.0071x over previous
import functools

import jax
import jax.numpy as jnp
from jax import lax
from jax.experimental import pallas as pl
from jax.experimental.pallas import tpu as pltpu
from jax.experimental.pallas import tpu_sc as plsc

_NMS_THR = 0.3
_MAX_OUT = 320
_SPEC_K = 4


def _nms_body(scores_ref, x1_ref, y1_ref, x2_ref, y2_ref, keep_ref, s_ref, a_ref):
    n_b = scores_ref.shape[0]
    rows, fcols = scores_ref.shape[1], scores_ref.shape[2]
    for b in range(n_b):
        x1 = x1_ref[b]
        y1 = y1_ref[b]
        x2 = x2_ref[b]
        y2 = y2_ref[b]
        a_ref[b] = jnp.maximum(x2 - x1, 0.0) * jnp.maximum(y2 - y1, 0.0)
        s_ref[b] = scores_ref[b]
    col = lax.broadcasted_iota(jnp.int32, (rows, fcols), 1)
    row = lax.broadcasted_iota(jnp.int32, (rows, fcols), 0)
    gidx = (col + fcols * row).astype(jnp.float32)
    nbig = jnp.float32(rows * fcols)

    def red(op, arr):
        return op(op(arr, axis=1, keepdims=True), axis=0, keepdims=True)

    def chain(b, m1):
        s = s_ref[b]
        a = a_ref[b]
        x1 = x1_ref[b]
        y1 = y1_ref[b]
        x2 = x2_ref[b]
        y2 = y2_ref[b]

        cands = []
        s_cur = s
        m = m1
        for g in range(_SPEC_K):
            act = m > -0.5
            idx = red(jnp.min, jnp.where(s_cur == m, gidx, nbig))
            cand = (gidx == idx) & act
            cands.append((cand, act))
            s_cur = jnp.where(cand, -3.0, s_cur)
            if g + 1 < _SPEC_K:
                m = red(jnp.max, s_cur)

        coords = []
        for cand, act in cands:
            cf = cand.astype(jnp.float32)
            coords.append((red(jnp.sum, x1 * cf), red(jnp.sum, y1 * cf),
                           red(jnp.sum, x2 * cf), red(jnp.sum, y2 * cf),
                           red(jnp.sum, a * cf)))

        def pair_iou(i, j):
            ix1, iy1, ix2, iy2, ia = coords[i]
            jx1, jy1, jx2, jy2, ja = coords[j]
            w = jnp.maximum(jnp.minimum(ix2, jx2) - jnp.maximum(ix1, jx1), 0.0)
            h = jnp.maximum(jnp.minimum(iy2, jy2) - jnp.maximum(iy1, jy1), 0.0)
            inter = w * h
            return inter / jnp.maximum(ia + ja - inter, 1e-9)

        commit = [cands[0][1]]
        for g in range(1, _SPEC_K):
            sup = commit[0] & (pair_iou(0, g) > _NMS_THR)
            for h in range(1, g):
                sup = sup | (commit[h] & (pair_iou(h, g) > _NMS_THR))
            commit.append(cands[g][1] & jnp.logical_not(sup))

        supp = None
        commit_mask = None
        for g in range(_SPEC_K):
            gx1, gy1, gx2, gy2, ga = coords[g]
            w = jnp.maximum(jnp.minimum(x2, gx2) - jnp.maximum(x1, gx1), 0.0)
            h = jnp.maximum(jnp.minimum(y2, gy2) - jnp.maximum(y1, gy1), 0.0)
            inter = w * h
            iou = inter / jnp.maximum(ga + a - inter, 1e-9)
            sg = (iou > _NMS_THR) & commit[g]
            cg = cands[g][0] & commit[g]
            supp = sg if supp is None else (supp | sg)
            commit_mask = cg if commit_mask is None else (commit_mask | cg)

        s_new = jnp.where(commit_mask, -2.0, jnp.where(supp, -1.0, s))
        s_ref[b] = s_new
        return red(jnp.max, s_new)

    def body(carry):
        return tuple(chain(b, carry[b]) for b in range(n_b))

    def cond(carry):
        m = carry[0]
        for b in range(1, n_b):
            m = jnp.maximum(m, carry[b])
        return m[0, 0] > -0.5

    init = tuple(red(jnp.max, s_ref[b]) for b in range(n_b))
    lax.while_loop(cond, body, init)
    for b in range(n_b):
        keep_ref[b] = (s_ref[b] == -2.0).astype(jnp.float32)


def _compact_body(keep_hbm, cls_hbm, x1_hbm, y1_hbm, x2_hbm, y2_hbm, sc_hbm,
                  oc_hbm, o1_hbm, o2_hbm, o3_hbm, o4_hbm, os_hbm,
                  k_s, c_s, x1_s, y1_s, x2_s, y2_s, s_s,
                  oc_s, o1_s, o2_s, o3_s, o4_s, os_s):
    nb = keep_hbm.shape[0]
    npad = keep_hbm.shape[1]
    wid = lax.axis_index("s") * 2 + lax.axis_index("c")

    @pl.when(wid < nb)
    def _():
        pltpu.sync_copy(keep_hbm.at[wid], k_s)
        pltpu.sync_copy(cls_hbm.at[wid], c_s)
        pltpu.sync_copy(x1_hbm.at[wid], x1_s)
        pltpu.sync_copy(y1_hbm.at[wid], y1_s)
        pltpu.sync_copy(x2_hbm.at[wid], x2_s)
        pltpu.sync_copy(y2_hbm.at[wid], y2_s)
        pltpu.sync_copy(sc_hbm.at[wid], s_s)

        outs = (oc_s, o1_s, o2_s, o3_s, o4_s, os_s)
        srcs = (c_s, x1_s, y1_s, x2_s, y2_s, s_s)

        def zero(i, _):
            z = jnp.zeros((16,), jnp.float32)
            for oref in outs:
                oref[pl.ds(i * 16, 16)] = z
            return 0

        lax.fori_loop(0, _MAX_OUT // 16, zero, 0)

        def step(i, base):
            kv = k_s[pl.ds(i * 16, 16)]
            ci = plsc.cumsum(kv)
            pos = base + ci.astype(jnp.int32) - 1
            msk = (kv > 0.5) & (pos < _MAX_OUT)
            for src, dst in zip(srcs, outs):
                plsc.store_scatter(dst, [pos], src[pl.ds(i * 16, 16)], mask=msk)
            return base + jnp.sum(kv).astype(jnp.int32)

        lax.fori_loop(0, npad // 16, step, jnp.int32(0))

        pltpu.sync_copy(oc_s, oc_hbm.at[wid])
        pltpu.sync_copy(o1_s, o1_hbm.at[wid])
        pltpu.sync_copy(o2_s, o2_hbm.at[wid])
        pltpu.sync_copy(o3_s, o3_hbm.at[wid])
        pltpu.sync_copy(o4_s, o4_hbm.at[wid])
        pltpu.sync_copy(os_s, os_hbm.at[wid])


@functools.partial(jax.jit, static_argnums=(5,))
def _run_nms(scores8, x18, y18, x28, y28, shape3):
    return pl.pallas_call(
        _nms_body,
        out_shape=jax.ShapeDtypeStruct(shape3, jnp.float32),
        scratch_shapes=[
            pltpu.VMEM(shape3, jnp.float32),
            pltpu.VMEM(shape3, jnp.float32),
        ],
    )(scores8, x18, y18, x28, y28)


@functools.partial(jax.jit, static_argnums=(7, 8))
def _run_compact(keep, cls_a, x1, y1, x2, y2, sc_a, nb, npad):
    mesh = plsc.VectorSubcoreMesh(core_axis_name="c", subcore_axis_name="s")
    out_type = [jax.ShapeDtypeStruct((nb, _MAX_OUT), jnp.float32)] * 6
    scratch = [pltpu.VMEM((npad,), jnp.float32)] * 7 + \
              [pltpu.VMEM((_MAX_OUT,), jnp.float32)] * 6
    return pl.kernel(
        _compact_body,
        out_type=out_type,
        mesh=mesh,
        scratch_types=scratch,
        compiler_params=pltpu.CompilerParams(needs_layout_passes=False),
    )(keep, cls_a, x1, y1, x2, y2, sc_a)


def kernel(cls_proposals, gt_classes, box_proposals, gt_boxes, proposal_scores):
    nb = gt_boxes.shape[0]
    cls_all = jnp.concatenate([gt_classes, cls_proposals], axis=1)
    box_all = jnp.concatenate([gt_boxes, box_proposals], axis=1)
    sc_all = jnp.concatenate([gt_classes, proposal_scores], axis=1)
    n = box_all.shape[1]
    npad = ((n + 511) // 512) * 512

    x1 = box_all[:, :, 0]
    y1 = box_all[:, :, 1]
    x2 = box_all[:, :, 2]
    y2 = box_all[:, :, 3]

    fcols = npad // 8
    shape3 = (nb, 8, fcols)

    def fold(arr, value):
        out = jnp.full((nb, npad), value, jnp.float32)
        out = out.at[:, :n].set(arr)
        return out.reshape(shape3)

    scores8 = fold(sc_all, -1.0)
    x18 = fold(x1, 0.0)
    y18 = fold(y1, 0.0)
    x28 = fold(x2, 0.0)
    y28 = fold(y2, 0.0)

    keep = _run_nms(scores8, x18, y18, x28, y28, shape3).reshape(nb, npad)

    def pad_cols(arr):
        return jnp.pad(arr, ((0, 0), (0, npad - n)))

    oc, o1, o2, o3, o4, osc = _run_compact(
        keep, pad_cols(cls_all), pad_cols(x1), pad_cols(y1), pad_cols(x2),
        pad_cols(y2), pad_cols(sc_all), nb, npad)

    outb = jnp.stack([o1, o2, o3, o4], axis=-1)
    return oc, outb, osc
